# Initial kernel scaffold; baseline (speedup 1.0000x reference)
import jax
import jax.numpy as jnp
from jax.experimental import pallas as pl


def kernel(x, edge_index, W, b, a):
    raise NotImplementedError("write your pallas kernel here")


# speedup vs baseline: 9.6117x; 9.6117x over previous
import functools

import jax
import jax.numpy as jnp
from jax import lax
from jax.experimental import pallas as pl
from jax.experimental.pallas import tpu as pltpu
from jax.experimental.pallas import tpu_sc as plsc

N = 10000
E = 320000
IN_DIM = 128
OUT_DIM = 32

NW = 32
CHUNK = 128
EP = ((E + NW * CHUNK - 1) // (NW * CHUNK)) * (NW * CHUNK)
EPT = EP // NW
C = EPT // CHUNK
NPAD = 10112
RPT = NPAD // 16


def _mlp_body(x_ref, wt_ref, b_ref, a_ref, h_ref):
    xb = x_ref[...]
    h = jnp.dot(xb, wt_ref[...], preferred_element_type=jnp.float32)
    h = h + b_ref[...]
    aa = a_ref[0, 0]
    h = jnp.where(h >= 0.0, h, aa * h)
    ss = jnp.sum(h * h, axis=1, keepdims=True)
    nrm = jnp.sqrt(ss)
    h_ref[...] = h / jnp.maximum(nrm, 1e-12)


def _mlp(x, wt, b2, a2):
    ROWS = 2000
    grid = N // ROWS
    return pl.pallas_call(
        _mlp_body,
        grid=(grid,),
        in_specs=[
            pl.BlockSpec((ROWS, IN_DIM), lambda i: (i, 0)),
            pl.BlockSpec((IN_DIM, OUT_DIM), lambda i: (0, 0)),
            pl.BlockSpec((1, OUT_DIM), lambda i: (0, 0)),
            pl.BlockSpec(memory_space=pltpu.SMEM),
        ],
        out_specs=pl.BlockSpec((ROWS, OUT_DIM), lambda i: (i, 0)),
        out_shape=jax.ShapeDtypeStruct((N, OUT_DIM), jnp.float32),
    )(x, wt, b2, a2)


def _agg_body(src_hbm, dst_hbm, h_hbm, zrows_hbm, zcnt_hbm,
              agg_out, cnt_out,
              sidx_v, didx_v, rows_v, ones_v, cntz_v, agg_sh, cnt_sh, sem):
    cid = lax.axis_index("c")
    sid = lax.axis_index("s")
    wid = cid * 16 + sid

    for i in range(CHUNK // 16):
        ones_v[pl.ds(i * 16, 16)] = jnp.ones((16,), jnp.float32)

    pltpu.sync_copy(zrows_hbm.at[pl.ds(sid * RPT, RPT)],
                    agg_sh.at[pl.ds(sid * RPT, RPT)])
    pltpu.sync_copy(zcnt_hbm.at[pl.ds(sid * RPT, RPT)], cntz_v)
    pltpu.sync_copy(cntz_v, cnt_sh.at[pl.ds(sid * RPT, RPT)])
    plsc.subcore_barrier()

    pltpu.sync_copy(src_hbm.at[wid], sidx_v)
    pltpu.sync_copy(dst_hbm.at[wid], didx_v)

    def body(j, carry):
        pltpu.async_copy(h_hbm.at[sidx_v.at[j]], rows_v, sem).wait()
        pltpu.sync_copy(rows_v, agg_sh.at[didx_v.at[j]], add=True)
        pltpu.sync_copy(ones_v, cnt_sh.at[didx_v.at[j]], add=True)
        return carry

    lax.fori_loop(0, C, body, 0)
    plsc.subcore_barrier()

    pltpu.sync_copy(agg_sh.at[pl.ds(sid * RPT, RPT)],
                    agg_out.at[cid, pl.ds(sid * RPT, RPT)])
    pltpu.sync_copy(cnt_sh.at[pl.ds(sid * RPT, RPT)], cntz_v)
    pltpu.sync_copy(cntz_v, cnt_out.at[cid, sid, 0])


_agg = functools.partial(
    pl.kernel,
    out_type=[jax.ShapeDtypeStruct((2, NPAD, OUT_DIM), jnp.float32),
              jax.ShapeDtypeStruct((2, 16, 1, RPT), jnp.float32)],
    mesh=plsc.VectorSubcoreMesh(core_axis_name="c", subcore_axis_name="s"),
    scratch_types=[
        pltpu.VMEM((C, CHUNK), jnp.int32),
        pltpu.VMEM((C, CHUNK), jnp.int32),
        pltpu.VMEM((CHUNK, OUT_DIM), jnp.float32),
        pltpu.VMEM((CHUNK,), jnp.float32),
        pltpu.VMEM((RPT,), jnp.float32),
        pltpu.VMEM_SHARED((NPAD, OUT_DIM), jnp.float32),
        pltpu.VMEM_SHARED((NPAD,), jnp.float32),
        pltpu.SemaphoreType.DMA,
    ],
    compiler_params=pltpu.CompilerParams(use_tc_tiling_on_sc=False),
)(_agg_body)


def _comb_body(agg_ref, cnt_ref, out_ref):
    s = agg_ref[0] + agg_ref[1]
    c = cnt_ref[0] + cnt_ref[1]
    out_ref[...] = s / jnp.maximum(c, 1.0)


def _comb(agg, cnt3):
    ROWS = 2000
    grid = N // ROWS
    return pl.pallas_call(
        _comb_body,
        grid=(grid,),
        in_specs=[
            pl.BlockSpec((2, ROWS, OUT_DIM), lambda i: (0, i, 0)),
            pl.BlockSpec((2, ROWS, 1), lambda i: (0, i, 0)),
        ],
        out_specs=pl.BlockSpec((ROWS, OUT_DIM), lambda i: (i, 0)),
        out_shape=jax.ShapeDtypeStruct((N, OUT_DIM), jnp.float32),
    )(agg, cnt3)


def kernel(x, edge_index, W, b, a):
    h = _mlp(x, W.T, b.reshape(1, OUT_DIM),
             jnp.asarray(a, jnp.float32).reshape(1, 1))

    pad = EP - E
    srcp = jnp.concatenate(
        [edge_index[0], jnp.zeros((pad,), jnp.int32)]).reshape(NW, C, CHUNK)
    dstp = jnp.concatenate(
        [edge_index[1], jnp.full((pad,), N, jnp.int32)]).reshape(NW, C, CHUNK)
    zrows = jnp.zeros((NPAD, OUT_DIM), jnp.float32)
    zcnt = jnp.zeros((NPAD,), jnp.float32)

    agg, cnt = _agg(srcp, dstp, h, zrows, zcnt)
    x_neigh = _comb(agg, cnt.reshape(2, NPAD, 1))
    return (h, x_neigh)
